# Initial kernel scaffold; baseline (speedup 1.0000x reference)
import functools

import jax
import jax.numpy as jnp
from jax import lax
from jax.experimental import pallas as pl
from jax.experimental.pallas import tpu as pltpu
from jax.experimental.pallas import tpu_sc as plsc

DIM = 128
ROWS = 4096 * 50
NC = 2
NS = 16
NW = NC * NS
BPW = ROWS // NW
GSZ = 128
NG_CH = 5
CH = GSZ * NG_CH
NCH = BPW // CH

_mesh = plsc.VectorSubcoreMesh(core_axis_name="c", subcore_axis_name="s")


@functools.partial(
    pl.kernel,
    mesh=_mesh,
    out_type=jax.ShapeDtypeStruct((ROWS, DIM), jnp.float32),
    scratch_types=[
        pltpu.VMEM((NG_CH, GSZ), jnp.int32),
        pltpu.VMEM((CH, DIM), jnp.float32),
        pltpu.SemaphoreType.DMA,
    ],
)
def _gather_kernel(idx_hbm, table_hbm, out_hbm, idx_v, rows_v, sem):
    wid = lax.axis_index("s") * NC + lax.axis_index("c")
    base_irow = wid * (BPW // GSZ)

    def chunk(c, carry):
        irow = base_irow + c * NG_CH
        pltpu.sync_copy(idx_hbm.at[pl.ds(irow, NG_CH)], idx_v)
        copies = [
            pltpu.async_copy(
                table_hbm.at[idx_v.at[j]],
                rows_v.at[pl.ds(j * GSZ, GSZ)],
                sem,
            )
            for j in range(NG_CH)
        ]
        for cp in copies:
            cp.wait()
        pltpu.sync_copy(rows_v, out_hbm.at[pl.ds(irow * GSZ, CH)])
        return carry

    lax.fori_loop(0, NCH, chunk, 0)


def kernel(input, emb_weight):
    idx2d = input.reshape(ROWS // GSZ, GSZ).astype(jnp.int32)
    out = _gather_kernel(idx2d, emb_weight)
    return out.reshape(input.shape[0], input.shape[1], DIM)


# speedup vs baseline: 3.2733x; 3.2733x over previous
import functools

import jax
import jax.numpy as jnp
from jax import lax
from jax.experimental import pallas as pl
from jax.experimental.pallas import tpu as pltpu
from jax.experimental.pallas import tpu_sc as plsc

DIM = 128
ROWS = 4096 * 50
NC = 2
NS = 16
NW = NC * NS
BPW = ROWS // NW
GSZ = 128
NG_CH = 5
CH = GSZ * NG_CH
NCH = BPW // CH

_mesh = plsc.VectorSubcoreMesh(core_axis_name="c", subcore_axis_name="s")


@functools.partial(
    pl.kernel,
    mesh=_mesh,
    out_type=jax.ShapeDtypeStruct((ROWS, DIM), jnp.float32),
    scratch_types=[
        pltpu.VMEM((CH,), jnp.int32),
        pltpu.VMEM((CH, DIM), jnp.float32),
        pltpu.SemaphoreType.DMA,
    ],
)
def _gather_kernel(idx_hbm, table_hbm, out_hbm, idx_v, rows_v, sem):
    wid = lax.axis_index("s") * NC + lax.axis_index("c")
    base = wid * BPW

    def chunk(c, carry):
        off = base + c * CH
        pltpu.sync_copy(idx_hbm.at[pl.ds(off, CH)], idx_v)
        copies = [
            pltpu.async_copy(
                table_hbm.at[idx_v.at[pl.ds(j * GSZ, GSZ)]],
                rows_v.at[pl.ds(j * GSZ, GSZ)],
                sem,
            )
            for j in range(NG_CH)
        ]
        for cp in copies:
            cp.wait()
        pltpu.sync_copy(rows_v, out_hbm.at[pl.ds(off, CH)])
        return carry

    lax.fori_loop(0, NCH, chunk, 0)


def kernel(input, emb_weight):
    idx1d = input.reshape(ROWS).astype(jnp.int32)
    out = _gather_kernel(idx1d, emb_weight)
    return out.reshape(input.shape[0], input.shape[1], DIM)


# speedup vs baseline: 3.3001x; 1.0082x over previous
import functools

import jax
import jax.numpy as jnp
from jax import lax
from jax.experimental import pallas as pl
from jax.experimental.pallas import tpu as pltpu
from jax.experimental.pallas import tpu_sc as plsc

DIM = 128
ROWS = 4096 * 50
NC = 2
NS = 16
NW = NC * NS
BPW = ROWS // NW
GSZ = 80
NG_CH = 5
CH = GSZ * NG_CH
NPAIR = BPW // (2 * CH)

_mesh = plsc.VectorSubcoreMesh(core_axis_name="c", subcore_axis_name="s")


@functools.partial(
    pl.kernel,
    mesh=_mesh,
    out_type=jax.ShapeDtypeStruct((ROWS, DIM), jnp.float32),
    scratch_types=[
        pltpu.VMEM((CH,), jnp.int32),
        pltpu.VMEM((CH,), jnp.int32),
        pltpu.VMEM((CH, DIM), jnp.float32),
        pltpu.VMEM((CH, DIM), jnp.float32),
        pltpu.SemaphoreType.DMA,
        pltpu.SemaphoreType.DMA,
        pltpu.SemaphoreType.DMA,
        pltpu.SemaphoreType.DMA,
    ],
)
def _gather_kernel(idx_hbm, table_hbm, out_hbm, idx_a, idx_b, rows_a,
                   rows_b, gsem_a, gsem_b, ssem_a, ssem_b):
    wid = lax.axis_index("s") * NC + lax.axis_index("c")
    base = wid * BPW

    def fire_gathers(idx_v, rows_v, gsem):
        return [
            pltpu.async_copy(
                table_hbm.at[idx_v.at[pl.ds(j * GSZ, GSZ)]],
                rows_v.at[pl.ds(j * GSZ, GSZ)],
                gsem,
            )
            for j in range(NG_CH)
        ]

    def pair(g, carry):
        off_a = base + g * (2 * CH)
        off_b = off_a + CH
        pltpu.sync_copy(idx_hbm.at[pl.ds(off_a, CH)], idx_a)
        ga = fire_gathers(idx_a, rows_a, gsem_a)
        pltpu.sync_copy(idx_hbm.at[pl.ds(off_b, CH)], idx_b)
        gb = fire_gathers(idx_b, rows_b, gsem_b)
        for cp in ga:
            cp.wait()
        sa = pltpu.async_copy(rows_a, out_hbm.at[pl.ds(off_a, CH)], ssem_a)
        for cp in gb:
            cp.wait()
        sb = pltpu.async_copy(rows_b, out_hbm.at[pl.ds(off_b, CH)], ssem_b)
        sa.wait()
        sb.wait()
        return carry

    lax.fori_loop(0, NPAIR, pair, 0)


def kernel(input, emb_weight):
    idx1d = input.reshape(ROWS).astype(jnp.int32)
    out = _gather_kernel(idx1d, emb_weight)
    return out.reshape(input.shape[0], input.shape[1], DIM)
